# baseline (speedup 1.0000x reference)
import functools

import jax
import jax.numpy as jnp
from jax import lax
from jax.experimental import pallas as pl
from jax.experimental.pallas import tpu as pltpu
from jax.experimental.pallas import tpu_sc as plsc

N = 10000
E = 320000
D = 128
C = 40
NP = 10240
NC = 2
NS = 16
NW = NC * NS
B = 128
NCH = 80
NBUF = 4
EB = 64
ECH = 160
ECH0 = 76
ECH1 = 244
TCH = NS * (ECH0 + ECH1)
EPT = NCH * B
EPAD = NW * EPT
ROWS_PT = NP // NS
DEG_PT = 3 * NP // NS

_sc_mesh = plsc.VectorSubcoreMesh(core_axis_name="c", subcore_axis_name="s")


@functools.partial(
    pl.kernel,
    out_type=jax.ShapeDtypeStruct((NC * 3 * NP,), jnp.float32),
    mesh=_sc_mesh,
    scratch_types=[
        [pltpu.VMEM((B,), jnp.int32) for _ in range(NBUF)],
        [pltpu.SemaphoreType.DMA for _ in range(NBUF)],
        [pltpu.SemaphoreType.DMA for _ in range(NBUF)],
        pltpu.VMEM((B,), jnp.float32),
        pltpu.VMEM((DEG_PT,), jnp.float32),
        pltpu.VMEM_SHARED((3 * NP,), jnp.float32),
    ],
)
def _deg_kernel(dsts_hbm, out_hbm, dstc, si, ss, ones, zbuf, acc):
    cid = lax.axis_index("c")
    sid = lax.axis_index("s")
    w = cid * NS + sid
    for j in range(B // 16):
        ones[pl.ds(j * 16, 16)] = jnp.ones((16,), jnp.float32)
    for j in range(DEG_PT // 16):
        zbuf[pl.ds(j * 16, 16)] = jnp.zeros((16,), jnp.float32)
    pltpu.sync_copy(zbuf, acc.at[pl.ds(sid * DEG_PT, DEG_PT)])
    plsc.subcore_barrier()

    def start_idx(ch, b):
        pltpu.async_copy(dsts_hbm.at[w * 3 * NCH + ch], dstc[b], si[b])

    def wait_idx(b):
        pltpu.make_async_copy(dsts_hbm.at[0], dstc[b], si[b]).wait()

    for b in range(NBUF):
        start_idx(b, b)

    NQ = 3 * NCH // NBUF

    def quad(q, refill):
        for b in range(NBUF):
            wait_idx(b)
            pltpu.sync_copy(ones, acc.at[dstc[b]], add=True)
            if refill:
                start_idx((q + 1) * NBUF + b, b)

    lax.fori_loop(0, NQ - 1, lambda q, c: (quad(q, True), c)[1], 0)
    quad(NQ - 1, False)

    plsc.subcore_barrier()
    pltpu.sync_copy(acc.at[pl.ds(sid * DEG_PT, DEG_PT)],
                    out_hbm.at[pl.ds(cid * 3 * NP + sid * DEG_PT, DEG_PT)])


@functools.partial(
    pl.kernel,
    out_type=jax.ShapeDtypeStruct((NC * 3 * NP, D), jnp.float32),
    mesh=_sc_mesh,
    scratch_types=[
        [pltpu.VMEM((EB,), jnp.int32) for _ in range(NBUF)],
        [pltpu.VMEM((EB,), jnp.int32) for _ in range(NBUF)],
        [pltpu.VMEM((EB, D), jnp.float32) for _ in range(NBUF)],
        [pltpu.SemaphoreType.DMA for _ in range(NBUF)],
        [pltpu.SemaphoreType.DMA for _ in range(NBUF)],
        [pltpu.SemaphoreType.DMA for _ in range(NBUF)],
        [pltpu.SemaphoreType.DMA for _ in range(NBUF)],
        pltpu.VMEM_SHARED((NP, D), jnp.float32),
    ],
)
def _edge_kernel(srcs_hbm, dsts_hbm, g_hbm, out_hbm,
                 srcc, dstc, rows, ssi, si, sg, ss, acc):
    cid = lax.axis_index("c")
    sid = lax.axis_index("s")
    w = cid * NS + sid

    def start_gather(b):
        pltpu.async_copy(g_hbm.at[srcc[b]], rows[b], sg[b])

    def wait_gather(b):
        pltpu.make_async_copy(g_hbm.at[pl.ds(0, EB)], rows[b], sg[b]).wait()

    def sync_scatter(b):
        pltpu.sync_copy(rows[b], acc.at[dstc[b]], add=True)

    nq = jnp.where(cid == 0, ECH0 // NBUF, ECH1 // NBUF)
    tile_row0 = jnp.where(cid == 0, sid * ECH0, NS * ECH0 + sid * ECH1)

    for i in range(3):
        row0 = i * TCH + tile_row0

        def start_idx(ch, b):
            pltpu.async_copy(srcs_hbm.at[row0 + ch], srcc[b], ssi[b])
            pltpu.async_copy(dsts_hbm.at[row0 + ch], dstc[b], si[b])

        def wait_sidx(b):
            pltpu.make_async_copy(srcs_hbm.at[0], srcc[b], ssi[b]).wait()

        def wait_didx(b):
            pltpu.make_async_copy(dsts_hbm.at[0], dstc[b], si[b]).wait()

        pltpu.sync_copy(g_hbm.at[pl.ds(i * NP + sid * ROWS_PT, ROWS_PT)],
                        acc.at[pl.ds(sid * ROWS_PT, ROWS_PT)])
        plsc.subcore_barrier()

        for b in range(NBUF):
            start_idx(b, b)
        for b in range(NBUF):
            wait_sidx(b)
            start_gather(b)

        def quad(q, refill):
            for b in range(NBUF):
                wait_gather(b)
                wait_didx(b)
                sync_scatter(b)
                if refill:
                    start_idx((q + 1) * NBUF + b, b)
            if refill:
                for b in range(NBUF):
                    wait_sidx(b)
                    start_gather(b)

        lax.fori_loop(0, nq - 1, lambda q, c: (quad(q, True), c)[1], 0)
        quad(nq - 1, False)

        plsc.subcore_barrier()
        pltpu.sync_copy(acc.at[pl.ds(sid * ROWS_PT, ROWS_PT)],
                        out_hbm.at[pl.ds((cid * 3 + i) * NP + sid * ROWS_PT, ROWS_PT)])


RB = 1024


def _mm_body(x_ref, w_ref, dinvb_ref, g_ref):
    h = jnp.dot(x_ref[...], w_ref[...], preferred_element_type=jnp.float32)
    for i in range(3):
        g_ref[i] = h[:, i * D:(i + 1) * D] * dinvb_ref[i]


_mm_call = pl.pallas_call(
    _mm_body,
    grid=(NP // RB,),
    in_specs=[
        pl.BlockSpec((RB, D), lambda r: (r, 0)),
        pl.BlockSpec((D, 3 * D), lambda r: (0, 0)),
        pl.BlockSpec((3, RB, D), lambda r: (0, r, 0)),
    ],
    out_specs=pl.BlockSpec((3, RB, D), lambda r: (0, r, 0)),
    out_shape=jax.ShapeDtypeStruct((3, NP, D), jnp.float32),
)


def _comb_body(p_ref, g_ref, dinvb_ref, bc_ref, wl_ref, bl_ref, o_ref):
    s = p_ref[0] + p_ref[1] - g_ref[...]
    s = s * dinvb_ref[...]
    bc = bc_ref[...]
    wl = wl_ref[...]
    xs = [jnp.maximum(s[i] + bc[i], 0.0) for i in range(3)]
    mx = jnp.maximum(xs[0], jnp.maximum(xs[1], xs[2]))
    mn = jnp.minimum(xs[0], jnp.minimum(xs[1], xs[2]))
    mean = (xs[0] + xs[1] + xs[2]) * (1.0 / 3.0)
    out = jnp.dot(mx, wl[0:D], preferred_element_type=jnp.float32)
    out = out + jnp.dot(mn, wl[D:2 * D], preferred_element_type=jnp.float32)
    out = out + jnp.dot(mean, wl[2 * D:3 * D], preferred_element_type=jnp.float32)
    out = out + bl_ref[...]
    z = out - jnp.max(out, axis=1, keepdims=True)
    o_ref[...] = z - jnp.log(jnp.sum(jnp.exp(z), axis=1, keepdims=True))


_comb_call = pl.pallas_call(
    _comb_body,
    grid=(NP // RB,),
    in_specs=[
        pl.BlockSpec((2, 3, RB, D), lambda r: (0, 0, r, 0)),
        pl.BlockSpec((3, RB, D), lambda r: (0, r, 0)),
        pl.BlockSpec((3, RB, D), lambda r: (0, r, 0)),
        pl.BlockSpec((3, D), lambda r: (0, 0)),
        pl.BlockSpec((3 * D, C), lambda r: (0, 0)),
        pl.BlockSpec((1, C), lambda r: (0, 0)),
    ],
    out_specs=pl.BlockSpec((RB, C), lambda r: (r, 0)),
    out_shape=jax.ShapeDtypeStruct((NP, C), jnp.float32),
)


def kernel(x, edge_index1, edge_index2, edge_index3, W1, b1, W2, b2, W3, b3, Wl, bl):
    xp = jnp.concatenate([x, jnp.zeros((NP - N, D), jnp.float32)], axis=0)
    Wcat = jnp.concatenate([W1, W2, W3], axis=1)
    bcat = jnp.stack([b1, b2, b3])

    pad = jnp.full((EPAD - E,), NP - 1, jnp.int32)
    srcs, dsts = [], []
    for i, ei in enumerate((edge_index1, edge_index2, edge_index3)):
        srcs.append(jnp.concatenate([ei[0], pad]) + i * NP)
        dsts.append(jnp.concatenate([ei[1], pad]))
    srcs_e = jnp.stack(srcs).reshape(3 * NW * ECH, EB)
    dsts_e = jnp.stack(dsts).reshape(3 * NW * ECH, EB)
    dsts_d = (jnp.stack(dsts) + jnp.arange(3, dtype=jnp.int32)[:, None] * NP)
    dsts_d = dsts_d.reshape(3, NW, EPT).transpose(1, 0, 2).reshape(NW * 3 * NCH, B)

    deg_p = _deg_kernel(dsts_d).reshape(NC, 3 * NP)
    dinv = lax.rsqrt(deg_p[0] + deg_p[1] + 1.0).reshape(3, NP)
    dinvb = jnp.broadcast_to(dinv[:, :, None], (3, NP, D))

    g3 = _mm_call(xp, Wcat, dinvb)
    part = _edge_kernel(srcs_e, dsts_e, g3.reshape(3 * NP, D))
    part = part.reshape(NC, 3, NP, D)
    out = _comb_call(part, g3, dinvb, bcat, Wl, bl.reshape(1, C))
    return out[:N]


# speedup vs baseline: 1.2454x; 1.2454x over previous
import functools

import jax
import jax.numpy as jnp
from jax import lax
from jax.experimental import pallas as pl
from jax.experimental.pallas import tpu as pltpu
from jax.experimental.pallas import tpu_sc as plsc

N = 10000
E = 320000
D = 128
C = 40
NP = 10240
NC = 2
NS = 16
NW = NC * NS
B = 128
NCH = 80
NBUF = 4
EB = 64
ECH = 160
ECH0 = 244
ECH1 = 76
TCH = NS * (ECH0 + ECH1)
EPT = NCH * B
EPAD = NW * EPT
ROWS_PT = NP // NS
DEG_PT = 3 * NP // NS

_sc_mesh = plsc.VectorSubcoreMesh(core_axis_name="c", subcore_axis_name="s")


@functools.partial(
    pl.kernel,
    out_type=jax.ShapeDtypeStruct((NC * 3 * NP,), jnp.float32),
    mesh=_sc_mesh,
    scratch_types=[
        [pltpu.VMEM((B,), jnp.int32) for _ in range(NBUF)],
        [pltpu.SemaphoreType.DMA for _ in range(NBUF)],
        [pltpu.SemaphoreType.DMA for _ in range(NBUF)],
        pltpu.VMEM((B,), jnp.float32),
        pltpu.VMEM((DEG_PT,), jnp.float32),
        pltpu.VMEM_SHARED((3 * NP,), jnp.float32),
    ],
)
def _deg_kernel(dsts_hbm, out_hbm, dstc, si, ss, ones, zbuf, acc):
    cid = lax.axis_index("c")
    sid = lax.axis_index("s")
    w = cid * NS + sid
    for j in range(B // 16):
        ones[pl.ds(j * 16, 16)] = jnp.ones((16,), jnp.float32)
    for j in range(DEG_PT // 16):
        zbuf[pl.ds(j * 16, 16)] = jnp.zeros((16,), jnp.float32)
    pltpu.sync_copy(zbuf, acc.at[pl.ds(sid * DEG_PT, DEG_PT)])
    plsc.subcore_barrier()

    def start_idx(ch, b):
        pltpu.async_copy(dsts_hbm.at[w * 3 * NCH + ch], dstc[b], si[b])

    def wait_idx(b):
        pltpu.make_async_copy(dsts_hbm.at[0], dstc[b], si[b]).wait()

    for b in range(NBUF):
        start_idx(b, b)

    NQ = 3 * NCH // NBUF

    def quad(q, refill):
        for b in range(NBUF):
            wait_idx(b)
            pltpu.sync_copy(ones, acc.at[dstc[b]], add=True)
            if refill:
                start_idx((q + 1) * NBUF + b, b)

    lax.fori_loop(0, NQ - 1, lambda q, c: (quad(q, True), c)[1], 0)
    quad(NQ - 1, False)

    plsc.subcore_barrier()
    pltpu.sync_copy(acc.at[pl.ds(sid * DEG_PT, DEG_PT)],
                    out_hbm.at[pl.ds(cid * 3 * NP + sid * DEG_PT, DEG_PT)])


@functools.partial(
    pl.kernel,
    out_type=jax.ShapeDtypeStruct((NC * 3 * NP, D), jnp.float32),
    mesh=_sc_mesh,
    scratch_types=[
        [pltpu.VMEM((EB,), jnp.int32) for _ in range(NBUF)],
        [pltpu.VMEM((EB,), jnp.int32) for _ in range(NBUF)],
        [pltpu.VMEM((EB, D), jnp.float32) for _ in range(NBUF)],
        [pltpu.SemaphoreType.DMA for _ in range(NBUF)],
        [pltpu.SemaphoreType.DMA for _ in range(NBUF)],
        [pltpu.SemaphoreType.DMA for _ in range(NBUF)],
        [pltpu.SemaphoreType.DMA for _ in range(NBUF)],
        pltpu.VMEM_SHARED((NP, D), jnp.float32),
    ],
)
def _edge_kernel(srcs_hbm, dsts_hbm, g_hbm, out_hbm,
                 srcc, dstc, rows, ssi, si, sg, ss, acc):
    cid = lax.axis_index("c")
    sid = lax.axis_index("s")
    w = cid * NS + sid

    def start_gather(b):
        pltpu.async_copy(g_hbm.at[srcc[b]], rows[b], sg[b])

    def wait_gather(b):
        pltpu.make_async_copy(g_hbm.at[pl.ds(0, EB)], rows[b], sg[b]).wait()

    def sync_scatter(b):
        pltpu.sync_copy(rows[b], acc.at[dstc[b]], add=True)

    nq = jnp.where(cid == 0, ECH0 // NBUF, ECH1 // NBUF)
    tile_row0 = jnp.where(cid == 0, sid * ECH0, NS * ECH0 + sid * ECH1)

    for i in range(3):
        row0 = i * TCH + tile_row0

        def start_idx(ch, b):
            pltpu.async_copy(srcs_hbm.at[row0 + ch], srcc[b], ssi[b])
            pltpu.async_copy(dsts_hbm.at[row0 + ch], dstc[b], si[b])

        def wait_sidx(b):
            pltpu.make_async_copy(srcs_hbm.at[0], srcc[b], ssi[b]).wait()

        def wait_didx(b):
            pltpu.make_async_copy(dsts_hbm.at[0], dstc[b], si[b]).wait()

        pltpu.sync_copy(g_hbm.at[pl.ds(i * NP + sid * ROWS_PT, ROWS_PT)],
                        acc.at[pl.ds(sid * ROWS_PT, ROWS_PT)])
        plsc.subcore_barrier()

        for b in range(NBUF):
            start_idx(b, b)
        for b in range(NBUF):
            wait_sidx(b)
            start_gather(b)

        def quad(q, refill):
            for b in range(NBUF):
                wait_gather(b)
                wait_didx(b)
                sync_scatter(b)
                if refill:
                    start_idx((q + 1) * NBUF + b, b)
            if refill:
                for b in range(NBUF):
                    wait_sidx(b)
                    start_gather(b)

        lax.fori_loop(0, nq - 1, lambda q, c: (quad(q, True), c)[1], 0)
        quad(nq - 1, False)

        plsc.subcore_barrier()
        pltpu.sync_copy(acc.at[pl.ds(sid * ROWS_PT, ROWS_PT)],
                        out_hbm.at[pl.ds((cid * 3 + i) * NP + sid * ROWS_PT, ROWS_PT)])


RB = 1024


def _mm_body(x_ref, w_ref, dinvb_ref, g_ref):
    h = jnp.dot(x_ref[...], w_ref[...], preferred_element_type=jnp.float32)
    for i in range(3):
        g_ref[i] = h[:, i * D:(i + 1) * D] * dinvb_ref[i]


_mm_call = pl.pallas_call(
    _mm_body,
    grid=(NP // RB,),
    in_specs=[
        pl.BlockSpec((RB, D), lambda r: (r, 0)),
        pl.BlockSpec((D, 3 * D), lambda r: (0, 0)),
        pl.BlockSpec((3, RB, D), lambda r: (0, r, 0)),
    ],
    out_specs=pl.BlockSpec((3, RB, D), lambda r: (0, r, 0)),
    out_shape=jax.ShapeDtypeStruct((3, NP, D), jnp.float32),
)


def _comb_body(p_ref, g_ref, dinvb_ref, bc_ref, wl_ref, bl_ref, o_ref):
    s = p_ref[0] + p_ref[1] - g_ref[...]
    s = s * dinvb_ref[...]
    bc = bc_ref[...]
    wl = wl_ref[...]
    xs = [jnp.maximum(s[i] + bc[i], 0.0) for i in range(3)]
    mx = jnp.maximum(xs[0], jnp.maximum(xs[1], xs[2]))
    mn = jnp.minimum(xs[0], jnp.minimum(xs[1], xs[2]))
    mean = (xs[0] + xs[1] + xs[2]) * (1.0 / 3.0)
    out = jnp.dot(mx, wl[0:D], preferred_element_type=jnp.float32)
    out = out + jnp.dot(mn, wl[D:2 * D], preferred_element_type=jnp.float32)
    out = out + jnp.dot(mean, wl[2 * D:3 * D], preferred_element_type=jnp.float32)
    out = out + bl_ref[...]
    z = out - jnp.max(out, axis=1, keepdims=True)
    o_ref[...] = z - jnp.log(jnp.sum(jnp.exp(z), axis=1, keepdims=True))


_comb_call = pl.pallas_call(
    _comb_body,
    grid=(NP // RB,),
    in_specs=[
        pl.BlockSpec((2, 3, RB, D), lambda r: (0, 0, r, 0)),
        pl.BlockSpec((3, RB, D), lambda r: (0, r, 0)),
        pl.BlockSpec((3, RB, D), lambda r: (0, r, 0)),
        pl.BlockSpec((3, D), lambda r: (0, 0)),
        pl.BlockSpec((3 * D, C), lambda r: (0, 0)),
        pl.BlockSpec((1, C), lambda r: (0, 0)),
    ],
    out_specs=pl.BlockSpec((RB, C), lambda r: (r, 0)),
    out_shape=jax.ShapeDtypeStruct((NP, C), jnp.float32),
)


def kernel(x, edge_index1, edge_index2, edge_index3, W1, b1, W2, b2, W3, b3, Wl, bl):
    xp = jnp.concatenate([x, jnp.zeros((NP - N, D), jnp.float32)], axis=0)
    Wcat = jnp.concatenate([W1, W2, W3], axis=1)
    bcat = jnp.stack([b1, b2, b3])

    pad = jnp.full((EPAD - E,), NP - 1, jnp.int32)
    srcs, dsts = [], []
    for i, ei in enumerate((edge_index1, edge_index2, edge_index3)):
        srcs.append(jnp.concatenate([ei[0], pad]) + i * NP)
        dsts.append(jnp.concatenate([ei[1], pad]))
    srcs_e = jnp.stack(srcs).reshape(3 * NW * ECH, EB)
    dsts_e = jnp.stack(dsts).reshape(3 * NW * ECH, EB)
    dsts_d = (jnp.stack(dsts) + jnp.arange(3, dtype=jnp.int32)[:, None] * NP)
    dsts_d = dsts_d.reshape(3, NW, EPT).transpose(1, 0, 2).reshape(NW * 3 * NCH, B)

    deg_p = _deg_kernel(dsts_d).reshape(NC, 3 * NP)
    dinv = lax.rsqrt(deg_p[0] + deg_p[1] + 1.0).reshape(3, NP)
    dinvb = jnp.broadcast_to(dinv[:, :, None], (3, NP, D))

    g3 = _mm_call(xp, Wcat, dinvb)
    part = _edge_kernel(srcs_e, dsts_e, g3.reshape(3 * NP, D))
    part = part.reshape(NC, 3, NP, D)
    out = _comb_call(part, g3, dinvb, bcat, Wl, bl.reshape(1, C))
    return out[:N]


# speedup vs baseline: 1.2491x; 1.0030x over previous
import functools

import jax
import jax.numpy as jnp
from jax import lax
from jax.experimental import pallas as pl
from jax.experimental.pallas import tpu as pltpu
from jax.experimental.pallas import tpu_sc as plsc

N = 10000
E = 320000
D = 128
C = 40
NP = 10240
NC = 2
NS = 16
NW = NC * NS
B = 128
NCH = 80
NBUF = 4
EB = 64
ECH = 160
ECH0 = 244
ECH1 = 76
TCH = NS * (ECH0 + ECH1)
EPT = NCH * B
EPAD = NW * EPT
ROWS_PT = NP // NS
DEG_PT = 3 * NP // NS

_sc_mesh = plsc.VectorSubcoreMesh(core_axis_name="c", subcore_axis_name="s")


@functools.partial(
    pl.kernel,
    out_type=jax.ShapeDtypeStruct((NC * 3 * NP,), jnp.float32),
    mesh=_sc_mesh,
    scratch_types=[
        [pltpu.VMEM((B,), jnp.int32) for _ in range(NBUF)],
        [pltpu.SemaphoreType.DMA for _ in range(NBUF)],
        [pltpu.SemaphoreType.DMA for _ in range(NBUF)],
        pltpu.VMEM((B,), jnp.float32),
        pltpu.VMEM((DEG_PT,), jnp.float32),
        pltpu.VMEM_SHARED((3 * NP,), jnp.float32),
    ],
)
def _deg_kernel(dsts_hbm, out_hbm, dstc, si, ss, ones, zbuf, acc):
    cid = lax.axis_index("c")
    sid = lax.axis_index("s")
    w = cid * NS + sid
    for j in range(B // 16):
        ones[pl.ds(j * 16, 16)] = jnp.ones((16,), jnp.float32)
    for j in range(DEG_PT // 16):
        zbuf[pl.ds(j * 16, 16)] = jnp.zeros((16,), jnp.float32)
    pltpu.sync_copy(zbuf, acc.at[pl.ds(sid * DEG_PT, DEG_PT)])
    plsc.subcore_barrier()

    def start_idx(ch, b):
        pltpu.async_copy(dsts_hbm.at[w * 3 * NCH + ch], dstc[b], si[b])

    def wait_idx(b):
        pltpu.make_async_copy(dsts_hbm.at[0], dstc[b], si[b]).wait()

    for b in range(NBUF):
        start_idx(b, b)

    NQ = 3 * NCH // NBUF

    def quad(q, refill):
        for b in range(NBUF):
            wait_idx(b)
            pltpu.sync_copy(ones, acc.at[dstc[b]], add=True)
            if refill:
                start_idx((q + 1) * NBUF + b, b)

    lax.fori_loop(0, NQ - 1, lambda q, c: (quad(q, True), c)[1], 0)
    quad(NQ - 1, False)

    plsc.subcore_barrier()
    pltpu.sync_copy(acc.at[pl.ds(sid * DEG_PT, DEG_PT)],
                    out_hbm.at[pl.ds(cid * 3 * NP + sid * DEG_PT, DEG_PT)])


@functools.partial(
    pl.kernel,
    out_type=jax.ShapeDtypeStruct((NC * 3 * NP, D), jnp.float32),
    mesh=_sc_mesh,
    scratch_types=[
        [pltpu.VMEM((EB,), jnp.int32) for _ in range(NBUF)],
        [pltpu.VMEM((EB,), jnp.int32) for _ in range(NBUF)],
        [pltpu.VMEM((EB, D), jnp.float32) for _ in range(NBUF)],
        pltpu.VMEM((EB, D), jnp.float32),
        [pltpu.SemaphoreType.DMA for _ in range(NBUF)],
        [pltpu.SemaphoreType.DMA for _ in range(NBUF)],
        [pltpu.SemaphoreType.DMA for _ in range(NBUF)],
        [pltpu.SemaphoreType.DMA for _ in range(NBUF)],
        pltpu.VMEM_SHARED((NP, D), jnp.float32),
    ],
)
def _edge_kernel(srcs_hbm, dsts_hbm, g_hbm, out_hbm,
                 srcc, dstc, rows, zbuf, ssi, si, sg, ss, acc):
    cid = lax.axis_index("c")
    sid = lax.axis_index("s")
    w = cid * NS + sid

    def start_gather(b):
        pltpu.async_copy(g_hbm.at[srcc[b]], rows[b], sg[b])

    def wait_gather(b):
        pltpu.make_async_copy(g_hbm.at[pl.ds(0, EB)], rows[b], sg[b]).wait()

    def sync_scatter(b):
        pltpu.sync_copy(rows[b], acc.at[dstc[b]], add=True)

    nq = jnp.where(cid == 0, ECH0 // NBUF, ECH1 // NBUF)
    tile_row0 = jnp.where(cid == 0, sid * ECH0, NS * ECH0 + sid * ECH1)

    def zrow(r, carry):
        for j in range(D // 16):
            zbuf[r, pl.ds(j * 16, 16)] = jnp.zeros((16,), jnp.float32)
        return carry

    lax.fori_loop(0, EB, zrow, 0)

    for i in range(3):
        row0 = i * TCH + tile_row0

        def start_idx(ch, b):
            pltpu.async_copy(srcs_hbm.at[row0 + ch], srcc[b], ssi[b])
            pltpu.async_copy(dsts_hbm.at[row0 + ch], dstc[b], si[b])

        def wait_sidx(b):
            pltpu.make_async_copy(srcs_hbm.at[0], srcc[b], ssi[b]).wait()

        def wait_didx(b):
            pltpu.make_async_copy(dsts_hbm.at[0], dstc[b], si[b]).wait()

        for k in range(ROWS_PT // EB):
            pltpu.sync_copy(zbuf, acc.at[pl.ds(sid * ROWS_PT + k * EB, EB)])
        plsc.subcore_barrier()

        for b in range(NBUF):
            start_idx(b, b)
        for b in range(NBUF):
            wait_sidx(b)
            start_gather(b)

        def quad(q, refill):
            for b in range(NBUF):
                wait_gather(b)
                wait_didx(b)
                sync_scatter(b)
                if refill:
                    start_idx((q + 1) * NBUF + b, b)
            if refill:
                for b in range(NBUF):
                    wait_sidx(b)
                    start_gather(b)

        lax.fori_loop(0, nq - 1, lambda q, c: (quad(q, True), c)[1], 0)
        quad(nq - 1, False)

        plsc.subcore_barrier()
        pltpu.sync_copy(acc.at[pl.ds(sid * ROWS_PT, ROWS_PT)],
                        out_hbm.at[pl.ds((cid * 3 + i) * NP + sid * ROWS_PT, ROWS_PT)])


RB = 1024


def _mm_body(x_ref, w_ref, dinvb_ref, g_ref):
    h = jnp.dot(x_ref[...], w_ref[...], preferred_element_type=jnp.float32)
    for i in range(3):
        g_ref[i] = h[:, i * D:(i + 1) * D] * dinvb_ref[i]


_mm_call = pl.pallas_call(
    _mm_body,
    grid=(NP // RB,),
    in_specs=[
        pl.BlockSpec((RB, D), lambda r: (r, 0)),
        pl.BlockSpec((D, 3 * D), lambda r: (0, 0)),
        pl.BlockSpec((3, RB, D), lambda r: (0, r, 0)),
    ],
    out_specs=pl.BlockSpec((3, RB, D), lambda r: (0, r, 0)),
    out_shape=jax.ShapeDtypeStruct((3, NP, D), jnp.float32),
)


def _comb_body(p_ref, g_ref, dinvb_ref, bc_ref, wl_ref, bl_ref, o_ref):
    s = p_ref[0] + p_ref[1] + g_ref[...]
    s = s * dinvb_ref[...]
    bc = bc_ref[...]
    wl = wl_ref[...]
    xs = [jnp.maximum(s[i] + bc[i], 0.0) for i in range(3)]
    mx = jnp.maximum(xs[0], jnp.maximum(xs[1], xs[2]))
    mn = jnp.minimum(xs[0], jnp.minimum(xs[1], xs[2]))
    mean = (xs[0] + xs[1] + xs[2]) * (1.0 / 3.0)
    out = jnp.dot(mx, wl[0:D], preferred_element_type=jnp.float32)
    out = out + jnp.dot(mn, wl[D:2 * D], preferred_element_type=jnp.float32)
    out = out + jnp.dot(mean, wl[2 * D:3 * D], preferred_element_type=jnp.float32)
    out = out + bl_ref[...]
    z = out - jnp.max(out, axis=1, keepdims=True)
    o_ref[...] = z - jnp.log(jnp.sum(jnp.exp(z), axis=1, keepdims=True))


_comb_call = pl.pallas_call(
    _comb_body,
    grid=(NP // RB,),
    in_specs=[
        pl.BlockSpec((2, 3, RB, D), lambda r: (0, 0, r, 0)),
        pl.BlockSpec((3, RB, D), lambda r: (0, r, 0)),
        pl.BlockSpec((3, RB, D), lambda r: (0, r, 0)),
        pl.BlockSpec((3, D), lambda r: (0, 0)),
        pl.BlockSpec((3 * D, C), lambda r: (0, 0)),
        pl.BlockSpec((1, C), lambda r: (0, 0)),
    ],
    out_specs=pl.BlockSpec((RB, C), lambda r: (r, 0)),
    out_shape=jax.ShapeDtypeStruct((NP, C), jnp.float32),
)


def kernel(x, edge_index1, edge_index2, edge_index3, W1, b1, W2, b2, W3, b3, Wl, bl):
    xp = jnp.concatenate([x, jnp.zeros((NP - N, D), jnp.float32)], axis=0)
    Wcat = jnp.concatenate([W1, W2, W3], axis=1)
    bcat = jnp.stack([b1, b2, b3])

    pad = jnp.full((EPAD - E,), NP - 1, jnp.int32)
    srcs, dsts = [], []
    for i, ei in enumerate((edge_index1, edge_index2, edge_index3)):
        srcs.append(jnp.concatenate([ei[0], pad]) + i * NP)
        dsts.append(jnp.concatenate([ei[1], pad]))
    srcs_e = jnp.stack(srcs).reshape(3 * NW * ECH, EB)
    dsts_e = jnp.stack(dsts).reshape(3 * NW * ECH, EB)
    dsts_d = (jnp.stack(dsts) + jnp.arange(3, dtype=jnp.int32)[:, None] * NP)
    dsts_d = dsts_d.reshape(3, NW, EPT).transpose(1, 0, 2).reshape(NW * 3 * NCH, B)

    deg_p = _deg_kernel(dsts_d).reshape(NC, 3 * NP)
    dinv = lax.rsqrt(deg_p[0] + deg_p[1] + 1.0).reshape(3, NP)
    dinvb = jnp.broadcast_to(dinv[:, :, None], (3, NP, D))

    g3 = _mm_call(xp, Wcat, dinvb)
    part = _edge_kernel(srcs_e, dsts_e, g3.reshape(3 * NP, D))
    part = part.reshape(NC, 3, NP, D)
    out = _comb_call(part, g3, dinvb, bcat, Wl, bl.reshape(1, C))
    return out[:N]
